# Initial kernel scaffold; baseline (speedup 1.0000x reference)
import jax
import jax.numpy as jnp
from jax.experimental import pallas as pl


def kernel(x, gate_w, gate_b, w1, w3, w2):
    raise NotImplementedError("write your pallas kernel here")


# speedup vs baseline: 1.2752x; 1.2752x over previous
import functools

import jax
import jax.numpy as jnp
from jax import lax
from jax.experimental import pallas as pl
from jax.experimental.pallas import tpu as pltpu
from jax.experimental.pallas import tpu_sc as plsc

_B, _S, _D, _F, _E = 2, 4096, 768, 1024, 16
_N = _B * _S
_THRESH = 0.8


_TG = 512


def _gate_body(x_ref, w_ref, b_ref, o_ref):
  o_ref[...] = (
      jnp.dot(x_ref[...], w_ref[...], preferred_element_type=jnp.float32)
      + b_ref[...]
  )


def _gate(x_flat, gate_w, gate_b):
  return pl.pallas_call(
      _gate_body,
      grid=(_N // _TG,),
      in_specs=[
          pl.BlockSpec((_TG, _D), lambda i: (i, 0)),
          pl.BlockSpec((_D, _E), lambda i: (0, 0)),
          pl.BlockSpec((1, _E), lambda i: (0, 0)),
      ],
      out_specs=pl.BlockSpec((_TG, _E), lambda i: (i, 0)),
      out_shape=jax.ShapeDtypeStruct((_N, _E), jnp.float32),
  )(x_flat, gate_w, gate_b.reshape(1, _E))


try:
  _INFO = plsc.get_sparse_core_info()
  _NC, _NS, _L = _INFO.num_cores, _INFO.num_subcores, _INFO.num_lanes
except ValueError:
  _NC, _NS, _L = 2, 16, 16
_NW = _NC * _NS
_TPW = _N // _NW


def _router_body(logits_hbm, rw_hbm, probs_hbm, ac_hbm, log_v, rw_v, p_v,
                 ac_v):
  c = lax.axis_index("c")
  s = lax.axis_index("s")
  wid = s * _NC + c
  base = wid * _TPW
  pltpu.sync_copy(logits_hbm.at[pl.ds(base, _TPW), :], log_v)
  eidx = lax.iota(jnp.int32, _L)

  def group(g, _):
    acc = jnp.zeros((_L,), jnp.int32)
    for j in range(_L):
      i = g * _L + j
      lv = log_v[i, :]
      m = jnp.max(lv)
      ex = jnp.exp(lv - m)
      p = ex / jnp.sum(ex)
      p_v[i, :] = p
      sp, order = plsc.sort_key_val(p, eidx, descending=True)
      shifted = plsc.cumsum(sp) - sp
      act = shifted < _THRESH
      ap = jnp.where(act, sp, jnp.zeros_like(sp))
      aw = ap / (jnp.sum(ap) + 1e-6)
      plsc.store_scatter(rw_v.at[i], [order], aw)
      cnt = jnp.sum(act.astype(jnp.int32))
      acc = jnp.where(eidx == j, cnt, acc)
    ac_v[pl.ds(g * _L, _L)] = acc
    return 0

  lax.fori_loop(0, _TPW // _L, group, 0)
  pltpu.sync_copy(rw_v, rw_hbm.at[pl.ds(base, _TPW), :])
  pltpu.sync_copy(p_v, probs_hbm.at[pl.ds(base, _TPW), :])
  pltpu.sync_copy(ac_v, ac_hbm.at[pl.ds(base, _TPW)])


def _router(logits):
  f32 = jnp.float32
  return pl.kernel(
      _router_body,
      out_type=(
          jax.ShapeDtypeStruct((_N, _E), f32),
          jax.ShapeDtypeStruct((_N, _E), f32),
          jax.ShapeDtypeStruct((_N,), jnp.int32),
      ),
      mesh=plsc.VectorSubcoreMesh(
          core_axis_name="c", subcore_axis_name="s"
      ),
      compiler_params=pltpu.CompilerParams(needs_layout_passes=False),
      scratch_types=[
          pltpu.VMEM((_TPW, _E), f32),
          pltpu.VMEM((_TPW, _E), f32),
          pltpu.VMEM((_TPW, _E), f32),
          pltpu.VMEM((_TPW,), jnp.int32),
      ],
  )(logits)


def _loss_body(rw_ref, p_ref, lb_ref, le_ref, first_ref):
  rw = rw_ref[...]
  p = p_ref[...]
  mask = (rw > 0.0).astype(jnp.float32)
  tpe = jnp.sum(mask, axis=0)
  p_sum = jnp.sum(p, axis=0)
  lb = _E * jnp.sum((tpe / _N) * (p_sum / _N))
  lb_ref[...] = jnp.full((1, 1), lb, jnp.float32)
  le = -jnp.sum(p * jnp.log(p + 1e-6)) / _N
  le_ref[...] = jnp.full((1, 1), le, jnp.float32)
  any_e = jnp.max(mask, axis=0)
  cand = jnp.where(any_e > 0.0, lax.iota(jnp.int32, _E), _E)
  fm = jnp.min(cand)
  first_ref[...] = jnp.full((1, 1), jnp.where(fm == _E, 0, fm), jnp.int32)


def _losses(rw, probs):
  return pl.pallas_call(
      _loss_body,
      grid=(1,),
      in_specs=[
          pl.BlockSpec((_N, _E), lambda i: (0, 0)),
          pl.BlockSpec((_N, _E), lambda i: (0, 0)),
      ],
      out_specs=[
          pl.BlockSpec((1, 1), lambda i: (0, 0)),
          pl.BlockSpec((1, 1), lambda i: (0, 0)),
          pl.BlockSpec((1, 1), lambda i: (0, 0)),
      ],
      out_shape=[
          jax.ShapeDtypeStruct((1, 1), jnp.float32),
          jax.ShapeDtypeStruct((1, 1), jnp.float32),
          jax.ShapeDtypeStruct((1, 1), jnp.int32),
      ],
  )(rw, probs)


_TT = 512


def _mlp_body(first_ref, x_ref, w1_ref, w3_ref, w2_ref, rw_ref, o_ref):
  xb = x_ref[...]
  h1 = jnp.dot(xb, w1_ref[0], preferred_element_type=jnp.float32)
  h3 = jnp.dot(xb, w3_ref[0], preferred_element_type=jnp.float32)
  h = h1 * jax.nn.sigmoid(h1) * h3
  out = jnp.dot(h, w2_ref[0], preferred_element_type=jnp.float32)
  lane = lax.broadcasted_iota(jnp.int32, (_TT, _E), 1)
  scale = jnp.sum(
      jnp.where(lane == first_ref[0], rw_ref[...], 0.0), axis=1,
      keepdims=True)
  o_ref[...] = out * scale


def _mlp(first, x_flat, w1, w3, w2, rw):
  grid_spec = pltpu.PrefetchScalarGridSpec(
      num_scalar_prefetch=1,
      grid=(_N // _TT,),
      in_specs=[
          pl.BlockSpec((_TT, _D), lambda i, f: (i, 0)),
          pl.BlockSpec((1, _D, _F), lambda i, f: (f[0], 0, 0)),
          pl.BlockSpec((1, _D, _F), lambda i, f: (f[0], 0, 0)),
          pl.BlockSpec((1, _F, _D), lambda i, f: (f[0], 0, 0)),
          pl.BlockSpec((_TT, _E), lambda i, f: (i, 0)),
      ],
      out_specs=pl.BlockSpec((_TT, _D), lambda i, f: (i, 0)),
  )
  return pl.pallas_call(
      _mlp_body,
      grid_spec=grid_spec,
      out_shape=jax.ShapeDtypeStruct((_N, _D), jnp.float32),
  )(first, x_flat, w1, w3, w2, rw)


@jax.jit
def kernel(x, gate_w, gate_b, w1, w3, w2):
  x_flat = x.reshape(_N, _D)
  logits = _gate(x_flat, gate_w, gate_b)
  rw, probs, ac = _router(logits)
  lb, le, first = _losses(rw, probs)
  out = _mlp(first.reshape(1), x_flat, w1, w3, w2, rw)
  return (
      out.reshape(_B, _S, _D),
      lb.reshape(()),
      le.reshape(()),
      ac.reshape(_B, _S),
  )
